# baseline (speedup 1.0000x reference)
import functools

import jax
import jax.numpy as jnp
from jax import lax
from jax.experimental import pallas as pl
from jax.experimental.pallas import tpu as pltpu
from jax.experimental.pallas import tpu_sc as plsc

U_SIZE = 1000000
I_SIZE = 100000
DIM = 64
BATCH = 16384

NUM_CORES = 2
NUM_SUBCORES = 16
NUM_WORKERS = NUM_CORES * NUM_SUBCORES
ROWS_PER_WORKER = BATCH // NUM_WORKERS
STAGE_ROWS = BATCH + 128
CAP = BATCH
NIDX_VECS = BATCH // 16
LANES = 16
NREG = 16
FLUSH_AT = 113
NBUF = 2
WBLK = 2

_COMPILER_PARAMS = pltpu.CompilerParams(
    needs_layout_passes=False, use_tc_tiling_on_sc=True,
    disable_bounds_checks=True, disable_semaphore_checks=True)


def _lane0(v):
  return lax.squeeze(lax.slice(v, (0,), (1,)), dimensions=(0,))


def _lane(v, i):
  return lax.squeeze(lax.slice(v, (i,), (i + 1,)), dimensions=(0,))


def _make_extract(n_rows):
  sfx = "u" if n_rows > 500000 else "i"
  nb = n_rows // 128
  ts = nb * 128
  tailn = n_rows - ts
  max_wblocks = -(-nb // NUM_WORKERS) + 1
  shift = max(0, (-(-max_wblocks // NREG) - 1).bit_length())
  mesh = plsc.VectorSubcoreMesh(core_axis_name="c", subcore_axis_name="s")

  @functools.partial(
      pl.kernel,
      mesh=mesh,
      out_type=jax.ShapeDtypeStruct((STAGE_ROWS, 2 * DIM), jnp.float32),
      compiler_params=_COMPILER_PARAMS,
      scratch_types=[
          pltpu.VMEM((BATCH,), jnp.int32),
          pltpu.VMEM((CAP + 16,), jnp.int32),
          pltpu.VMEM((CAP + 16,), jnp.int32),
          pltpu.VMEM((CAP,), jnp.int32),
          [pltpu.VMEM((64, WBLK * 128), jnp.float32) for _ in range(NBUF)],
          pltpu.VMEM((128, 2 * DIM), jnp.float32),
          pltpu.VMEM((2, 128), jnp.int32),
          [pltpu.SemaphoreType.DMA for _ in range(NBUF)],
          pltpu.SemaphoreType.DMA,
      ],
  )
  def k(idx_hbm, ut_hbm, tail_hbm, rows_hbm,
        idx_v, myu_v, mypos_v, bpos_v, vbufs, lrows, lpos_v,
        sems, semw):
    wid = lax.axis_index("s") * NUM_CORES + lax.axis_index("c")
    blk0 = (wid * nb) >> 5
    blk1 = ((wid + 1) * nb) >> 5
    is_last = wid == NUM_WORKERS - 1
    lanes = lax.iota(jnp.int32, LANES)
    safe_pos = jnp.full((LANES,), BATCH, jnp.int32)

    with jax.named_scope("ph_init_" + sfx):
      for j in range(2):
        for t in range(128 // 16):
          lpos_v[j, pl.ds(t * 16, 16)] = safe_pos
      pltpu.sync_copy(idx_hbm, idx_v)

    with jax.named_scope("ph_filter_" + sfx):
      def fbody(i, ptr_v):
        ptr = _lane0(ptr_v)
        uvec = idx_v[pl.ds(i * 16, 16)]
        q = lax.shift_right_logical(uvec, 7)
        m = (q >= blk0) & (q < blk1)
        m = m | (is_last & (uvec >= ts))
        plsc.store_compressed(myu_v.at[pl.ds(ptr, 16)], uvec, mask=m)
        plsc.store_compressed(mypos_v.at[pl.ds(ptr, 16)], i * 16 + lanes,
                              mask=m)
        return ptr_v + plsc.all_reduce_population_count(m)
      nmine_v = lax.fori_loop(0, NIDX_VECS, fbody,
                              jnp.zeros((LANES,), jnp.int32), unroll=False)
      nmine = _lane0(nmine_v)
      nvec = (nmine + 15) >> 4

    def region_of(uvec):
      r = lax.shift_right_logical(
          lax.shift_right_logical(uvec, 7) - blk0, shift)
      return jnp.minimum(r, NREG - 1)

    def cbody(v, cnts):
      uvec = myu_v[pl.ds(v * 16, 16)]
      valid = (v * 16 + lanes) < nmine
      r = region_of(uvec)
      for reg in range(NREG):
        pc = plsc.all_reduce_population_count((r == reg) & valid)
        cnts = cnts + jnp.where(lanes == reg, pc, 0)
      return cnts
    with jax.named_scope("ph_bucketA_" + sfx):
      cnts_v = lax.fori_loop(0, nvec, cbody, jnp.zeros((LANES,), jnp.int32),
                             unroll=False)
      starts0_v = plsc.cumsum(cnts_v) - cnts_v

    def bbody(v, starts):
      uvec = myu_v[pl.ds(v * 16, 16)]
      pvec = mypos_v[pl.ds(v * 16, 16)]
      valid = (v * 16 + lanes) < nmine
      r = region_of(uvec)
      for reg in range(NREG):
        m = (r == reg) & valid
        ptr = _lane(starts, reg)
        plsc.store_compressed(idx_v.at[pl.ds(ptr, 16)], uvec, mask=m)
        plsc.store_compressed(bpos_v.at[pl.ds(ptr, 16)], pvec, mask=m)
        pc = plsc.all_reduce_population_count(m)
        starts = starts + jnp.where(lanes == reg, pc, 0)
      return starts
    with jax.named_scope("ph_bucketB_" + sfx):
      lax.fori_loop(0, nvec, bbody, starts0_v, unroll=False)

    def flush(c):
      m_, chunk_ = c
      row = chunk_ & 1
      for t in range(128 // 16):
        plsc.store_scatter(lpos_v,
                           [jnp.full((LANES,), row, jnp.int32),
                            t * 16 + lanes],
                           safe_pos, mask=(t * 16 + lanes) >= m_)
      pltpu.async_copy(lrows, rows_hbm.at[lpos_v.at[row]], semw).wait()
      return 0, chunk_ + 1

    def extract_vector(vec_i, b, carry, vbuf, col_base, from_tail):
      m, chunk = carry
      uvec = idx_v[pl.ds(vec_i * 16, 16)]
      pvec = bpos_v[pl.ds(vec_i * 16, 16)]
      gidx = vec_i * 16 + lanes
      if from_tail:
        match = (gidx < nmine) & (uvec >= ts)
      else:
        match = (gidx < nmine) & (lax.shift_right_logical(uvec, 7) == b)
      mi = match.astype(jnp.int32)
      pc = _lane0(plsc.all_reduce_population_count(match))

      @pl.when(pc > 0)
      def _do():
        slot_v = m + plsc.cumsum(mi) - mi
        if from_tail:
          uloc_v = uvec - ts
        else:
          uloc_v = (uvec & 127) + col_base
        plsc.store_scatter(
            lpos_v,
            [jnp.full((LANES,), chunk & 1, jnp.int32), slot_v],
            pvec, mask=match)
        for kd in range(DIM):
          fk = (lanes + kd) & (DIM - 1)
          if from_tail:
            val = plsc.load_gather(vbuf, [uloc_v, fk], mask=match)
          else:
            val = plsc.load_gather(vbuf, [fk, uloc_v], mask=match)
          plsc.store_scatter(lrows, [slot_v, fk], val, mask=match)

      return lax.cond(m + pc >= FLUSH_AT, flush, lambda c: c,
                      (m + pc, chunk))

    def scan_block(b, vbuf, col_base, carry):
      reg = jnp.minimum(
          lax.shift_right_logical(b - blk0, shift), NREG - 1)
      rs = jnp.sum(jnp.where(lanes == reg, starts0_v, 0))
      re = rs + jnp.sum(jnp.where(lanes == reg, cnts_v, 0))

      def vloop(v, c_):
        return extract_vector(v, b, c_, vbuf, col_base, from_tail=False)
      return lax.fori_loop(rs >> 4, (re + 15) >> 4, vloop, carry,
                           unroll=False)

    def start_copy(g, o):
      b = blk0 + g * WBLK
      return pltpu.async_copy(
          ut_hbm.at[:, pl.ds(b * 128, WBLK * 128)], vbufs[o], sems[o])

    def wait_copy(o):
      pltpu.make_async_copy(ut_hbm.at[:, pl.ds(0, WBLK * 128)], vbufs[o],
                            sems[o]).wait()

    ngroups = (blk1 - blk0 + WBLK - 1) // WBLK

    with jax.named_scope("ph_sweep_" + sfx):
      for o in range(NBUF - 1):
        @pl.when(blk0 + o * WBLK < blk1)
        def _prime(o=o):
          start_copy(o, o)

      def ring_body(q, carry):
        for o in range(NBUF):
          g = q * NBUF + o
          b0 = blk0 + g * WBLK

          def process(c_, g=g, o=o, b0=b0):
            wait_copy(o)

            @pl.when(b0 + (NBUF - 1) * WBLK < blk1)
            def _prefetch():
              start_copy(g + NBUF - 1, (o + NBUF - 1) % NBUF)

            for s in range(WBLK):
              def scan_s(c2, s=s, o=o, b0=b0):
                return scan_block(b0 + s, vbufs[o], s * 128, c2)
              c_ = lax.cond(b0 + s < blk1, scan_s, lambda c2: c2, c_)
            return c_

          carry = lax.cond(b0 < blk1, process, lambda c_: c_, carry)
        return carry

      carry = lax.fori_loop(0, (ngroups + NBUF - 1) // NBUF, ring_body,
                            (0, 0), unroll=False)

    with jax.named_scope("ph_tail_" + sfx):
      @pl.when(is_last)
      def _tail_copy():
        pltpu.sync_copy(tail_hbm, vbufs[0].at[pl.ds(0, tailn), pl.ds(0, 128)])

      def tail_loop(v, c_):
        return extract_vector(v, 0, c_, vbufs[0], 0, from_tail=True)
      carry = lax.cond(
          is_last,
          lambda c_: lax.fori_loop(0, nvec, tail_loop, c_, unroll=False),
          lambda c_: c_,
          carry)

      m_fin, chunk_fin = carry

      @pl.when(m_fin > 0)
      def _final_flush():
        flush((m_fin, chunk_fin))

  return k


def _make_dot():
  mesh = plsc.VectorSubcoreMesh(core_axis_name="c", subcore_axis_name="s")
  chunk = 128
  n_chunks = ROWS_PER_WORKER // chunk

  @functools.partial(
      pl.kernel,
      mesh=mesh,
      out_type=jax.ShapeDtypeStruct((BATCH,), jnp.float32),
      compiler_params=_COMPILER_PARAMS,
      scratch_types=[
          pltpu.VMEM((chunk, 2 * DIM), jnp.float32),
          pltpu.VMEM((chunk, 2 * DIM), jnp.float32),
          pltpu.VMEM((ROWS_PER_WORKER,), jnp.float32),
          pltpu.SemaphoreType.DMA,
      ],
  )
  def k(rows_u_hbm, rows_i_hbm, out_hbm, ubuf, ibuf, out_v, sem):
    wid = lax.axis_index("s") * NUM_CORES + lax.axis_index("c")
    base = wid * ROWS_PER_WORKER
    lanes = lax.iota(jnp.int32, LANES)

    def chunk_body(c, _):
      row0 = base + c * chunk
      cu = pltpu.async_copy(rows_u_hbm.at[pl.ds(row0, chunk)], ubuf, sem)
      ci = pltpu.async_copy(rows_i_hbm.at[pl.ds(row0, chunk)], ibuf, sem)
      cu.wait()
      ci.wait()

      def group_body(g, _g):
        j_vec = g * 16 + lanes
        acc = jnp.zeros((16,), jnp.float32)
        for d in range(DIM):
          col = (lanes + d) & (DIM - 1)
          ug = plsc.load_gather(ubuf, [j_vec, col])
          ig = plsc.load_gather(ibuf, [j_vec, col])
          acc = acc + ug * ig
        out_v[pl.ds(c * chunk + g * 16, 16)] = acc
        return _g
      lax.fori_loop(0, chunk // 16, group_body, 0, unroll=False)
      return _

    lax.fori_loop(0, n_chunks, chunk_body, 0, unroll=False)
    pltpu.sync_copy(out_v, out_hbm.at[pl.ds(base, ROWS_PER_WORKER)])

  return k


_extract_u = _make_extract(U_SIZE)
_extract_i = _make_extract(I_SIZE)
_dot = _make_dot()

_U_TS = (U_SIZE // 128) * 128
_I_TS = (I_SIZE // 128) * 128


@jax.jit
def kernel(users, items, user_emb, item_emb):
  tail_u = jnp.pad(user_emb[_U_TS:], ((0, 0), (0, DIM)))
  tail_i = jnp.pad(item_emb[_I_TS:], ((0, 0), (0, DIM)))
  rows_u = _extract_u(users, user_emb.T, tail_u)
  rows_i = _extract_i(items, item_emb.T, tail_i)
  return _dot(rows_u, rows_i)


# speedup vs baseline: 1.0663x; 1.0663x over previous
import functools

import jax
import jax.numpy as jnp
from jax import lax
from jax.experimental import pallas as pl
from jax.experimental.pallas import tpu as pltpu
from jax.experimental.pallas import tpu_sc as plsc

U_SIZE = 1000000
I_SIZE = 100000
DIM = 64
BATCH = 16384

NUM_CORES = 2
NUM_SUBCORES = 16
NUM_WORKERS = NUM_CORES * NUM_SUBCORES
ROWS_PER_WORKER = BATCH // NUM_WORKERS
STAGE_ROWS = BATCH + 128
CAP = BATCH
NIDX_VECS = BATCH // 16
LANES = 16
NREG = 16
FLUSH_AT = 113
NBUF = 3
WBLK = 1

_COMPILER_PARAMS = pltpu.CompilerParams(
    needs_layout_passes=False, use_tc_tiling_on_sc=True,
    disable_bounds_checks=True, disable_semaphore_checks=True)


def _lane0(v):
  return lax.squeeze(lax.slice(v, (0,), (1,)), dimensions=(0,))


def _lane(v, i):
  return lax.squeeze(lax.slice(v, (i,), (i + 1,)), dimensions=(0,))


def _make_extract(n_rows):
  sfx = "u" if n_rows > 500000 else "i"
  nb = n_rows // 128
  ts = nb * 128
  tailn = n_rows - ts
  max_wblocks = -(-nb // NUM_WORKERS) + 1
  shift = max(0, (-(-max_wblocks // NREG) - 1).bit_length())
  mesh = plsc.VectorSubcoreMesh(core_axis_name="c", subcore_axis_name="s")

  @functools.partial(
      pl.kernel,
      mesh=mesh,
      out_type=jax.ShapeDtypeStruct((STAGE_ROWS, 2 * DIM), jnp.float32),
      compiler_params=_COMPILER_PARAMS,
      scratch_types=[
          pltpu.VMEM((BATCH,), jnp.int32),
          pltpu.VMEM((CAP + 16,), jnp.int32),
          pltpu.VMEM((CAP + 16,), jnp.int32),
          pltpu.VMEM((CAP,), jnp.int32),
          [pltpu.VMEM((64, WBLK * 128), jnp.float32) for _ in range(NBUF)],
          pltpu.VMEM((256, 2 * DIM), jnp.float32),
          pltpu.VMEM((2, 128), jnp.int32),
          [pltpu.SemaphoreType.DMA for _ in range(NBUF)],
          pltpu.SemaphoreType.DMA,
      ],
  )
  def k(idx_hbm, ut_hbm, tail_hbm, rows_hbm,
        idx_v, myu_v, mypos_v, bpos_v, vbufs, lrows, lpos_v,
        sems, semw):
    wid = lax.axis_index("s") * NUM_CORES + lax.axis_index("c")
    blk0 = (wid * nb) >> 5
    blk1 = ((wid + 1) * nb) >> 5
    is_last = wid == NUM_WORKERS - 1
    lanes = lax.iota(jnp.int32, LANES)
    safe_pos = jnp.full((LANES,), BATCH, jnp.int32)

    with jax.named_scope("ph_init_" + sfx):
      for j in range(2):
        for t in range(128 // 16):
          lpos_v[j, pl.ds(t * 16, 16)] = safe_pos
      pltpu.sync_copy(idx_hbm, idx_v)

    with jax.named_scope("ph_filter_" + sfx):
      def fbody(i, ptr_v):
        ptr = _lane0(ptr_v)
        uvec = idx_v[pl.ds(i * 16, 16)]
        q = lax.shift_right_logical(uvec, 7)
        m = (q >= blk0) & (q < blk1)
        m = m | (is_last & (uvec >= ts))
        plsc.store_compressed(myu_v.at[pl.ds(ptr, 16)], uvec, mask=m)
        plsc.store_compressed(mypos_v.at[pl.ds(ptr, 16)], i * 16 + lanes,
                              mask=m)
        return ptr_v + plsc.all_reduce_population_count(m)
      nmine_v = lax.fori_loop(0, NIDX_VECS, fbody,
                              jnp.zeros((LANES,), jnp.int32), unroll=False)
      nmine = _lane0(nmine_v)
      nvec = (nmine + 15) >> 4

    def region_of(uvec):
      r = lax.shift_right_logical(
          lax.shift_right_logical(uvec, 7) - blk0, shift)
      return jnp.minimum(r, NREG - 1)

    def cbody(v, cnts):
      uvec = myu_v[pl.ds(v * 16, 16)]
      valid = (v * 16 + lanes) < nmine
      r = region_of(uvec)
      for reg in range(NREG):
        pc = plsc.all_reduce_population_count((r == reg) & valid)
        cnts = cnts + jnp.where(lanes == reg, pc, 0)
      return cnts
    with jax.named_scope("ph_bucketA_" + sfx):
      cnts_v = lax.fori_loop(0, nvec, cbody, jnp.zeros((LANES,), jnp.int32),
                             unroll=False)
      starts0_v = plsc.cumsum(cnts_v) - cnts_v

    def bbody(v, starts):
      uvec = myu_v[pl.ds(v * 16, 16)]
      pvec = mypos_v[pl.ds(v * 16, 16)]
      valid = (v * 16 + lanes) < nmine
      r = region_of(uvec)
      for reg in range(NREG):
        m = (r == reg) & valid
        ptr = _lane(starts, reg)
        plsc.store_compressed(idx_v.at[pl.ds(ptr, 16)], uvec, mask=m)
        plsc.store_compressed(bpos_v.at[pl.ds(ptr, 16)], pvec, mask=m)
        pc = plsc.all_reduce_population_count(m)
        starts = starts + jnp.where(lanes == reg, pc, 0)
      return starts
    with jax.named_scope("ph_bucketB_" + sfx):
      lax.fori_loop(0, nvec, bbody, starts0_v, unroll=False)

    def drain_one():
      pltpu.make_async_copy(rows_hbm.at[pl.ds(0, 128)],
                            lrows.at[pl.ds(0, 128)], semw).wait()

    def flush(c):
      m_, chunk_ = c
      row = chunk_ & 1

      @pl.when(chunk_ >= 1)
      def _drain_prev():
        drain_one()

      for t in range(128 // 16):
        plsc.store_scatter(lpos_v,
                           [jnp.full((LANES,), row, jnp.int32),
                            t * 16 + lanes],
                           safe_pos, mask=(t * 16 + lanes) >= m_)
      pltpu.async_copy(lrows.at[pl.ds(row * 128, 128)],
                       rows_hbm.at[lpos_v.at[row]], semw)
      return 0, chunk_ + 1

    def extract_vector(vec_i, b, carry, vbuf, col_base, from_tail):
      m, chunk = carry
      uvec = idx_v[pl.ds(vec_i * 16, 16)]
      pvec = bpos_v[pl.ds(vec_i * 16, 16)]
      gidx = vec_i * 16 + lanes
      if from_tail:
        match = (gidx < nmine) & (uvec >= ts)
      else:
        match = (gidx < nmine) & (lax.shift_right_logical(uvec, 7) == b)
      mi = match.astype(jnp.int32)
      pc = _lane0(plsc.all_reduce_population_count(match))

      @pl.when(pc > 0)
      def _do():
        slot_v = m + plsc.cumsum(mi) - mi
        lslot_v = (chunk & 1) * 128 + slot_v
        if from_tail:
          uloc_v = uvec - ts
        else:
          uloc_v = (uvec & 127) + col_base
        plsc.store_scatter(
            lpos_v,
            [jnp.full((LANES,), chunk & 1, jnp.int32), slot_v],
            pvec, mask=match)
        for kd in range(DIM):
          fk = (lanes + kd) & (DIM - 1)
          if from_tail:
            val = plsc.load_gather(vbuf, [uloc_v, fk], mask=match)
          else:
            val = plsc.load_gather(vbuf, [fk, uloc_v], mask=match)
          plsc.store_scatter(lrows, [lslot_v, fk], val, mask=match)

      return lax.cond(m + pc >= FLUSH_AT, flush, lambda c: c,
                      (m + pc, chunk))

    def scan_block(b, vbuf, col_base, carry):
      reg = jnp.minimum(
          lax.shift_right_logical(b - blk0, shift), NREG - 1)
      rs = jnp.sum(jnp.where(lanes == reg, starts0_v, 0))
      re = rs + jnp.sum(jnp.where(lanes == reg, cnts_v, 0))

      def vloop(v, c_):
        return extract_vector(v, b, c_, vbuf, col_base, from_tail=False)
      return lax.fori_loop(rs >> 4, (re + 15) >> 4, vloop, carry,
                           unroll=False)

    def start_copy(g, o):
      b = blk0 + g * WBLK
      return pltpu.async_copy(
          ut_hbm.at[:, pl.ds(b * 128, WBLK * 128)], vbufs[o], sems[o])

    def wait_copy(o):
      pltpu.make_async_copy(ut_hbm.at[:, pl.ds(0, WBLK * 128)], vbufs[o],
                            sems[o]).wait()

    ngroups = (blk1 - blk0 + WBLK - 1) // WBLK

    with jax.named_scope("ph_sweep_" + sfx):
      for o in range(NBUF - 1):
        @pl.when(blk0 + o * WBLK < blk1)
        def _prime(o=o):
          start_copy(o, o)

      def ring_body(q, carry):
        for o in range(NBUF):
          g = q * NBUF + o
          b0 = blk0 + g * WBLK

          def process(c_, g=g, o=o, b0=b0):
            wait_copy(o)

            @pl.when(b0 + (NBUF - 1) * WBLK < blk1)
            def _prefetch():
              start_copy(g + NBUF - 1, (o + NBUF - 1) % NBUF)

            for s in range(WBLK):
              def scan_s(c2, s=s, o=o, b0=b0):
                return scan_block(b0 + s, vbufs[o], s * 128, c2)
              c_ = lax.cond(b0 + s < blk1, scan_s, lambda c2: c2, c_)
            return c_

          carry = lax.cond(b0 < blk1, process, lambda c_: c_, carry)
        return carry

      carry = lax.fori_loop(0, (ngroups + NBUF - 1) // NBUF, ring_body,
                            (0, 0), unroll=False)

    with jax.named_scope("ph_tail_" + sfx):
      @pl.when(is_last)
      def _tail_copy():
        pltpu.sync_copy(tail_hbm, vbufs[0].at[pl.ds(0, tailn), pl.ds(0, 128)])

      def tail_loop(v, c_):
        return extract_vector(v, 0, c_, vbufs[0], 0, from_tail=True)
      carry = lax.cond(
          is_last,
          lambda c_: lax.fori_loop(0, nvec, tail_loop, c_, unroll=False),
          lambda c_: c_,
          carry)

      m_fin, chunk_fin = carry

      @pl.when(m_fin > 0)
      def _final_flush():
        flush((m_fin, chunk_fin))

      total_chunks = chunk_fin + jnp.where(m_fin > 0, 1, 0)

      @pl.when(total_chunks >= 1)
      def _final_drain():
        drain_one()

  return k


def _make_dot():
  mesh = plsc.VectorSubcoreMesh(core_axis_name="c", subcore_axis_name="s")
  chunk = 128
  n_chunks = ROWS_PER_WORKER // chunk

  @functools.partial(
      pl.kernel,
      mesh=mesh,
      out_type=jax.ShapeDtypeStruct((BATCH,), jnp.float32),
      compiler_params=_COMPILER_PARAMS,
      scratch_types=[
          pltpu.VMEM((chunk, 2 * DIM), jnp.float32),
          pltpu.VMEM((chunk, 2 * DIM), jnp.float32),
          pltpu.VMEM((ROWS_PER_WORKER,), jnp.float32),
          pltpu.SemaphoreType.DMA,
      ],
  )
  def k(rows_u_hbm, rows_i_hbm, out_hbm, ubuf, ibuf, out_v, sem):
    wid = lax.axis_index("s") * NUM_CORES + lax.axis_index("c")
    base = wid * ROWS_PER_WORKER
    lanes = lax.iota(jnp.int32, LANES)

    def chunk_body(c, _):
      row0 = base + c * chunk
      cu = pltpu.async_copy(rows_u_hbm.at[pl.ds(row0, chunk)], ubuf, sem)
      ci = pltpu.async_copy(rows_i_hbm.at[pl.ds(row0, chunk)], ibuf, sem)
      cu.wait()
      ci.wait()

      def group_body(g, _g):
        j_vec = g * 16 + lanes
        acc = jnp.zeros((16,), jnp.float32)
        for d in range(DIM):
          col = (lanes + d) & (DIM - 1)
          ug = plsc.load_gather(ubuf, [j_vec, col])
          ig = plsc.load_gather(ibuf, [j_vec, col])
          acc = acc + ug * ig
        out_v[pl.ds(c * chunk + g * 16, 16)] = acc
        return _g
      lax.fori_loop(0, chunk // 16, group_body, 0, unroll=False)
      return _

    lax.fori_loop(0, n_chunks, chunk_body, 0, unroll=False)
    pltpu.sync_copy(out_v, out_hbm.at[pl.ds(base, ROWS_PER_WORKER)])

  return k


_extract_u = _make_extract(U_SIZE)
_extract_i = _make_extract(I_SIZE)
_dot = _make_dot()

_U_TS = (U_SIZE // 128) * 128
_I_TS = (I_SIZE // 128) * 128


@jax.jit
def kernel(users, items, user_emb, item_emb):
  tail_u = jnp.pad(user_emb[_U_TS:], ((0, 0), (0, DIM)))
  tail_i = jnp.pad(item_emb[_I_TS:], ((0, 0), (0, DIM)))
  rows_u = _extract_u(users, user_emb.T, tail_u)
  rows_i = _extract_i(items, item_emb.T, tail_i)
  return _dot(rows_u, rows_i)


# speedup vs baseline: 1.1627x; 1.0904x over previous
import functools

import jax
import jax.numpy as jnp
from jax import lax
from jax.experimental import pallas as pl
from jax.experimental.pallas import tpu as pltpu
from jax.experimental.pallas import tpu_sc as plsc

U_SIZE = 1000000
I_SIZE = 100000
DIM = 64
BATCH = 16384

NUM_CORES = 2
NUM_SUBCORES = 16
NUM_WORKERS = NUM_CORES * NUM_SUBCORES
ROWS_PER_WORKER = BATCH // NUM_WORKERS
STAGE_ROWS = BATCH + 128
CAP = BATCH
NIDX_VECS = BATCH // 16
LANES = 16
NREG = 16
FLUSH_AT = 113
NBUF = 3

_COMPILER_PARAMS = pltpu.CompilerParams(
    needs_layout_passes=False, use_tc_tiling_on_sc=True,
    disable_bounds_checks=True, disable_semaphore_checks=True)


def _lane0(v):
  return lax.squeeze(lax.slice(v, (0,), (1,)), dimensions=(0,))


def _lane(v, i):
  return lax.squeeze(lax.slice(v, (i,), (i + 1,)), dimensions=(0,))


def _make_extract():
  mesh = plsc.VectorSubcoreMesh(core_axis_name="c", subcore_axis_name="s")

  @functools.partial(
      pl.kernel,
      mesh=mesh,
      out_type=(jax.ShapeDtypeStruct((STAGE_ROWS, 2 * DIM), jnp.float32),
                jax.ShapeDtypeStruct((STAGE_ROWS, 2 * DIM), jnp.float32)),
      compiler_params=_COMPILER_PARAMS,
      scratch_types=[
          pltpu.VMEM((BATCH,), jnp.int32),
          pltpu.VMEM((CAP + 16,), jnp.int32),
          pltpu.VMEM((CAP + 16,), jnp.int32),
          pltpu.VMEM((CAP,), jnp.int32),
          [pltpu.VMEM((64, 128), jnp.float32) for _ in range(NBUF)],
          pltpu.VMEM((256, 2 * DIM), jnp.float32),
          pltpu.VMEM((2, 128), jnp.int32),
          [pltpu.SemaphoreType.DMA for _ in range(NBUF)],
          pltpu.SemaphoreType.DMA,
      ],
  )
  def k(users_hbm, items_hbm, ut_hbm, it_hbm, tail_u_hbm, tail_i_hbm,
        rows_u_hbm, rows_i_hbm,
        idx_v, myu_v, mypos_v, bpos_v, vbufs, lrows, lpos_v, sems, semw):
    wid = lax.axis_index("s") * NUM_CORES + lax.axis_index("c")
    is_last = wid == NUM_WORKERS - 1
    lanes = lax.iota(jnp.int32, LANES)
    safe_pos = jnp.full((LANES,), BATCH, jnp.int32)

    for j in range(2):
      for t in range(128 // 16):
        lpos_v[j, pl.ds(t * 16, 16)] = safe_pos

    def run_table(n_rows, idx_hbm, tbl_hbm, tail_hbm, rows_hbm, sfx):
      nb = n_rows // 128
      ts = nb * 128
      tailn = n_rows - ts
      max_wblocks = -(-nb // NUM_WORKERS) + 1
      shift = max(0, (-(-max_wblocks // NREG) - 1).bit_length())
      blk0 = (wid * nb) >> 5
      blk1 = ((wid + 1) * nb) >> 5

      with jax.named_scope("ph_init_" + sfx):
        pltpu.sync_copy(idx_hbm, idx_v)

      with jax.named_scope("ph_filter_" + sfx):
        def fbody(i, ptr_v):
          ptr = _lane0(ptr_v)
          uvec = idx_v[pl.ds(i * 16, 16)]
          q = lax.shift_right_logical(uvec, 7)
          m = (q >= blk0) & (q < blk1)
          m = m | (is_last & (uvec >= ts))
          plsc.store_compressed(myu_v.at[pl.ds(ptr, 16)], uvec, mask=m)
          plsc.store_compressed(mypos_v.at[pl.ds(ptr, 16)],
                                i * 16 + lanes, mask=m)
          return ptr_v + plsc.all_reduce_population_count(m)
        nmine_v = lax.fori_loop(0, NIDX_VECS, fbody,
                                jnp.zeros((LANES,), jnp.int32),
                                unroll=False)
        nmine = _lane0(nmine_v)
        nvec = (nmine + 15) >> 4

      def region_of(uvec):
        r = lax.shift_right_logical(
            lax.shift_right_logical(uvec, 7) - blk0, shift)
        return jnp.minimum(r, NREG - 1)

      def cbody(v, cnts):
        uvec = myu_v[pl.ds(v * 16, 16)]
        valid = (v * 16 + lanes) < nmine
        r = region_of(uvec)
        for reg in range(NREG):
          pc = plsc.all_reduce_population_count((r == reg) & valid)
          cnts = cnts + jnp.where(lanes == reg, pc, 0)
        return cnts
      with jax.named_scope("ph_bucketA_" + sfx):
        cnts_v = lax.fori_loop(0, nvec, cbody,
                               jnp.zeros((LANES,), jnp.int32),
                               unroll=False)
        starts0_v = plsc.cumsum(cnts_v) - cnts_v

      def bbody(v, starts):
        uvec = myu_v[pl.ds(v * 16, 16)]
        pvec = mypos_v[pl.ds(v * 16, 16)]
        valid = (v * 16 + lanes) < nmine
        r = region_of(uvec)
        for reg in range(NREG):
          m = (r == reg) & valid
          ptr = _lane(starts, reg)
          plsc.store_compressed(idx_v.at[pl.ds(ptr, 16)], uvec, mask=m)
          plsc.store_compressed(bpos_v.at[pl.ds(ptr, 16)], pvec, mask=m)
          pc = plsc.all_reduce_population_count(m)
          starts = starts + jnp.where(lanes == reg, pc, 0)
        return starts
      with jax.named_scope("ph_bucketB_" + sfx):
        lax.fori_loop(0, nvec, bbody, starts0_v, unroll=False)

      def drain_one():
        pltpu.make_async_copy(rows_hbm.at[pl.ds(0, 128)],
                              lrows.at[pl.ds(0, 128)], semw).wait()

      def flush(c):
        m_, chunk_ = c
        row = chunk_ & 1

        @pl.when(chunk_ >= 1)
        def _drain_prev():
          drain_one()

        for t in range(128 // 16):
          plsc.store_scatter(lpos_v,
                             [jnp.full((LANES,), row, jnp.int32),
                              t * 16 + lanes],
                             safe_pos, mask=(t * 16 + lanes) >= m_)
        pltpu.async_copy(lrows.at[pl.ds(row * 128, 128)],
                         rows_hbm.at[lpos_v.at[row]], semw)
        return 0, chunk_ + 1

      def extract_vector(vec_i, b, carry, vbuf, from_tail):
        m, chunk = carry
        uvec = idx_v[pl.ds(vec_i * 16, 16)]
        pvec = bpos_v[pl.ds(vec_i * 16, 16)]
        gidx = vec_i * 16 + lanes
        if from_tail:
          match = (gidx < nmine) & (uvec >= ts)
        else:
          match = (gidx < nmine) & (lax.shift_right_logical(uvec, 7) == b)
        mi = match.astype(jnp.int32)
        pc = _lane0(plsc.all_reduce_population_count(match))

        @pl.when(pc > 0)
        def _do():
          slot_v = m + plsc.cumsum(mi) - mi
          lslot_v = (chunk & 1) * 128 + slot_v
          if from_tail:
            uloc_v = uvec - ts
          else:
            uloc_v = uvec & 127
          plsc.store_scatter(
              lpos_v,
              [jnp.full((LANES,), chunk & 1, jnp.int32), slot_v],
              pvec, mask=match)
          for kd in range(DIM):
            fk = (lanes + kd) & (DIM - 1)
            if from_tail:
              val = plsc.load_gather(vbuf, [uloc_v, fk], mask=match)
            else:
              val = plsc.load_gather(vbuf, [fk, uloc_v], mask=match)
            plsc.store_scatter(lrows, [lslot_v, fk], val, mask=match)

        return lax.cond(m + pc >= FLUSH_AT, flush, lambda c: c,
                        (m + pc, chunk))

      def scan_block(b, vbuf, carry):
        reg = jnp.minimum(
            lax.shift_right_logical(b - blk0, shift), NREG - 1)
        rs = jnp.sum(jnp.where(lanes == reg, starts0_v, 0))
        re = rs + jnp.sum(jnp.where(lanes == reg, cnts_v, 0))

        def vloop(v, c_):
          return extract_vector(v, b, c_, vbuf, from_tail=False)
        return lax.fori_loop(rs >> 4, (re + 15) >> 4, vloop, carry,
                             unroll=False)

      def start_copy(b, o):
        return pltpu.async_copy(
            tbl_hbm.at[:, pl.ds(b * 128, 128)], vbufs[o], sems[o])

      def wait_copy(o):
        pltpu.make_async_copy(tbl_hbm.at[:, pl.ds(0, 128)], vbufs[o],
                              sems[o]).wait()

      with jax.named_scope("ph_sweep_" + sfx):
        for o in range(NBUF - 1):
          @pl.when(blk0 + o < blk1)
          def _prime(o=o):
            start_copy(blk0 + o, o)

        def ring_body(q, carry):
          for o in range(NBUF):
            b = blk0 + q * NBUF + o

            def process(c_, b=b, o=o):
              wait_copy(o)

              @pl.when(b + NBUF - 1 < blk1)
              def _prefetch():
                start_copy(b + NBUF - 1, (o + NBUF - 1) % NBUF)

              return scan_block(b, vbufs[o], c_)

            carry = lax.cond(b < blk1, process, lambda c_: c_, carry)
          return carry

        carry = lax.fori_loop(0, (blk1 - blk0 + NBUF - 1) // NBUF,
                              ring_body, (0, 0), unroll=False)

      with jax.named_scope("ph_tail_" + sfx):
        @pl.when(is_last)
        def _tail_copy():
          pltpu.sync_copy(tail_hbm,
                          vbufs[0].at[pl.ds(0, tailn), pl.ds(0, 128)])

        def tail_loop(v, c_):
          return extract_vector(v, 0, c_, vbufs[0], from_tail=True)
        carry = lax.cond(
            is_last,
            lambda c_: lax.fori_loop(0, nvec, tail_loop, c_, unroll=False),
            lambda c_: c_,
            carry)

        m_fin, chunk_fin = carry

        @pl.when(m_fin > 0)
        def _final_flush():
          flush((m_fin, chunk_fin))

        total_chunks = chunk_fin + jnp.where(m_fin > 0, 1, 0)

        @pl.when(total_chunks >= 1)
        def _final_drain():
          drain_one()

    run_table(U_SIZE, users_hbm, ut_hbm, tail_u_hbm, rows_u_hbm, "u")
    run_table(I_SIZE, items_hbm, it_hbm, tail_i_hbm, rows_i_hbm, "i")

  return k


def _make_dot():
  mesh = plsc.VectorSubcoreMesh(core_axis_name="c", subcore_axis_name="s")
  chunk = 128
  n_chunks = ROWS_PER_WORKER // chunk

  @functools.partial(
      pl.kernel,
      mesh=mesh,
      out_type=jax.ShapeDtypeStruct((BATCH,), jnp.float32),
      compiler_params=_COMPILER_PARAMS,
      scratch_types=[
          pltpu.VMEM((chunk, 2 * DIM), jnp.float32),
          pltpu.VMEM((chunk, 2 * DIM), jnp.float32),
          pltpu.VMEM((ROWS_PER_WORKER,), jnp.float32),
          pltpu.SemaphoreType.DMA,
      ],
  )
  def k(rows_u_hbm, rows_i_hbm, out_hbm, ubuf, ibuf, out_v, sem):
    wid = lax.axis_index("s") * NUM_CORES + lax.axis_index("c")
    base = wid * ROWS_PER_WORKER
    lanes = lax.iota(jnp.int32, LANES)

    def chunk_body(c, _):
      row0 = base + c * chunk
      cu = pltpu.async_copy(rows_u_hbm.at[pl.ds(row0, chunk)], ubuf, sem)
      ci = pltpu.async_copy(rows_i_hbm.at[pl.ds(row0, chunk)], ibuf, sem)
      cu.wait()
      ci.wait()

      def group_body(g, _g):
        j_vec = g * 16 + lanes
        acc = jnp.zeros((16,), jnp.float32)
        for d in range(DIM):
          col = (lanes + d) & (DIM - 1)
          ug = plsc.load_gather(ubuf, [j_vec, col])
          ig = plsc.load_gather(ibuf, [j_vec, col])
          acc = acc + ug * ig
        out_v[pl.ds(c * chunk + g * 16, 16)] = acc
        return _g
      lax.fori_loop(0, chunk // 16, group_body, 0, unroll=False)
      return _

    lax.fori_loop(0, n_chunks, chunk_body, 0, unroll=False)
    pltpu.sync_copy(out_v, out_hbm.at[pl.ds(base, ROWS_PER_WORKER)])

  return k


_extract = _make_extract()
_dot = _make_dot()

_U_TS = (U_SIZE // 128) * 128
_I_TS = (I_SIZE // 128) * 128


@jax.jit
def kernel(users, items, user_emb, item_emb):
  tail_u = jnp.pad(user_emb[_U_TS:], ((0, 0), (0, DIM)))
  tail_i = jnp.pad(item_emb[_I_TS:], ((0, 0), (0, DIM)))
  rows_u, rows_i = _extract(users, items, user_emb.T, item_emb.T,
                            tail_u, tail_i)
  return _dot(rows_u, rows_i)


# speedup vs baseline: 2.0602x; 1.7720x over previous
import functools

import jax
import jax.numpy as jnp
from jax import lax
from jax.experimental import pallas as pl
from jax.experimental.pallas import tpu as pltpu
from jax.experimental.pallas import tpu_sc as plsc

U_SIZE = 1000000
I_SIZE = 100000
DIM = 64
BATCH = 16384

NUM_CORES = 2
NUM_SUBCORES = 16
NUM_WORKERS = NUM_CORES * NUM_SUBCORES
ROWS_PER_WORKER = BATCH // NUM_WORKERS
SPAN = 1024
LIN_CHUNKS = 7
OVERFLOW_BASE = NUM_WORKERS * SPAN
STAGE_ROWS = OVERFLOW_BASE + BATCH + 128
POSMAP_LEN = OVERFLOW_BASE + 128
CAP = BATCH
NIDX_VECS = BATCH // 16
LANES = 16
NREG = 16
FLUSH_AT = 113
NBUF = 3

_COMPILER_PARAMS = pltpu.CompilerParams(
    needs_layout_passes=False, use_tc_tiling_on_sc=True,
    disable_bounds_checks=True, disable_semaphore_checks=True)


def _lane0(v):
  return lax.squeeze(lax.slice(v, (0,), (1,)), dimensions=(0,))


def _lane(v, i):
  return lax.squeeze(lax.slice(v, (i,), (i + 1,)), dimensions=(0,))


def _make_extract():
  mesh = plsc.VectorSubcoreMesh(core_axis_name="c", subcore_axis_name="s")

  @functools.partial(
      pl.kernel,
      mesh=mesh,
      out_type=(jax.ShapeDtypeStruct((STAGE_ROWS, 2 * DIM), jnp.float32),
                jax.ShapeDtypeStruct((POSMAP_LEN,), jnp.int32),
                jax.ShapeDtypeStruct((STAGE_ROWS, 2 * DIM), jnp.float32),
                jax.ShapeDtypeStruct((POSMAP_LEN,), jnp.int32)),
      compiler_params=_COMPILER_PARAMS,
      scratch_types=[
          pltpu.VMEM((BATCH,), jnp.int32),
          pltpu.VMEM((CAP + 16,), jnp.int32),
          pltpu.VMEM((CAP + 16,), jnp.int32),
          pltpu.VMEM((CAP,), jnp.int32),
          [pltpu.VMEM((64, 128), jnp.float32) for _ in range(NBUF)],
          pltpu.VMEM((256, 2 * DIM), jnp.float32),
          pltpu.VMEM((2, 128), jnp.int32),
          [pltpu.SemaphoreType.DMA for _ in range(NBUF)],
          pltpu.SemaphoreType.DMA,
      ],
  )
  def k(users_hbm, items_hbm, ut_hbm, it_hbm, tail_u_hbm, tail_i_hbm,
        rows_u_hbm, pmap_u_hbm, rows_i_hbm, pmap_i_hbm,
        idx_v, myu_v, mypos_v, bpos_v, vbufs, lrows, lpos_v, sems, semw):
    wid = lax.axis_index("s") * NUM_CORES + lax.axis_index("c")
    is_last = wid == NUM_WORKERS - 1
    lanes = lax.iota(jnp.int32, LANES)
    safe_pos = jnp.full((LANES,), BATCH, jnp.int32)

    for j in range(2):
      for t in range(128 // 16):
        lpos_v[j, pl.ds(t * 16, 16)] = safe_pos

    def run_table(n_rows, idx_hbm, tbl_hbm, tail_hbm, rows_hbm, pmap_hbm,
                  sfx):
      nb = n_rows // 128
      ts = nb * 128
      tailn = n_rows - ts
      max_wblocks = -(-nb // NUM_WORKERS) + 1
      shift = max(0, (-(-max_wblocks // NREG) - 1).bit_length())
      blk0 = (wid * nb) >> 5
      blk1 = ((wid + 1) * nb) >> 5

      with jax.named_scope("ph_init_" + sfx):
        pltpu.sync_copy(idx_hbm, idx_v)

      with jax.named_scope("ph_filter_" + sfx):
        def fbody(i, ptr_v):
          ptr = _lane0(ptr_v)
          uvec = idx_v[pl.ds(i * 16, 16)]
          q = lax.shift_right_logical(uvec, 7)
          m = (q >= blk0) & (q < blk1)
          m = m | (is_last & (uvec >= ts))
          plsc.store_compressed(myu_v.at[pl.ds(ptr, 16)], uvec, mask=m)
          plsc.store_compressed(mypos_v.at[pl.ds(ptr, 16)],
                                i * 16 + lanes, mask=m)
          return ptr_v + plsc.all_reduce_population_count(m)
        nmine_v = lax.fori_loop(0, NIDX_VECS, fbody,
                                jnp.zeros((LANES,), jnp.int32),
                                unroll=False)
        nmine = _lane0(nmine_v)
        nvec = (nmine + 15) >> 4

      def region_of(uvec):
        r = lax.shift_right_logical(
            lax.shift_right_logical(uvec, 7) - blk0, shift)
        return jnp.minimum(r, NREG - 1)

      def cbody(v, cnts):
        uvec = myu_v[pl.ds(v * 16, 16)]
        valid = (v * 16 + lanes) < nmine
        r = region_of(uvec)
        for reg in range(NREG):
          pc = plsc.all_reduce_population_count((r == reg) & valid)
          cnts = cnts + jnp.where(lanes == reg, pc, 0)
        return cnts
      with jax.named_scope("ph_bucketA_" + sfx):
        cnts_v = lax.fori_loop(0, nvec, cbody,
                               jnp.zeros((LANES,), jnp.int32),
                               unroll=False)
        starts0_v = plsc.cumsum(cnts_v) - cnts_v

      def bbody(v, starts):
        uvec = myu_v[pl.ds(v * 16, 16)]
        pvec = mypos_v[pl.ds(v * 16, 16)]
        valid = (v * 16 + lanes) < nmine
        r = region_of(uvec)
        for reg in range(NREG):
          m = (r == reg) & valid
          ptr = _lane(starts, reg)
          plsc.store_compressed(idx_v.at[pl.ds(ptr, 16)], uvec, mask=m)
          plsc.store_compressed(bpos_v.at[pl.ds(ptr, 16)], pvec, mask=m)
          pc = plsc.all_reduce_population_count(m)
          starts = starts + jnp.where(lanes == reg, pc, 0)
        return starts
      with jax.named_scope("ph_bucketB_" + sfx):
        lax.fori_loop(0, nvec, bbody, starts0_v, unroll=False)

      myspan = wid * SPAN

      def drain_one():
        pltpu.make_async_copy(rows_hbm.at[pl.ds(0, 128)],
                              lrows.at[pl.ds(0, 128)], semw).wait()
        pltpu.make_async_copy(pmap_hbm.at[pl.ds(0, 128)],
                              lpos_v.at[0], semw).wait()

      def flush(c):
        m_, chunk_, wtot_ = c
        row = chunk_ & 1

        @pl.when(chunk_ >= 1)
        def _drain_prev():
          drain_one()

        for t in range(128 // 16):
          plsc.store_scatter(lpos_v,
                             [jnp.full((LANES,), row, jnp.int32),
                              t * 16 + lanes],
                             safe_pos, mask=(t * 16 + lanes) >= m_)

        def linear_flush(_):
          off = pl.multiple_of(myspan + wtot_, 8)
          pltpu.async_copy(lrows.at[pl.ds(row * 128, 128)],
                           rows_hbm.at[pl.ds(off, 128)], semw)
          pltpu.async_copy(lpos_v.at[row],
                           pmap_hbm.at[pl.ds(off, 128)], semw)
          return 0

        def overflow_flush(_):
          for t in range(128 // 16):
            lpos_v[row, pl.ds(t * 16, 16)] = (
                lpos_v[row, pl.ds(t * 16, 16)] + OVERFLOW_BASE)
          pltpu.async_copy(lrows.at[pl.ds(row * 128, 128)],
                           rows_hbm.at[lpos_v.at[row]], semw)
          pltpu.async_copy(lpos_v.at[row],
                           pmap_hbm.at[pl.ds(OVERFLOW_BASE, 128)], semw)
          return 0

        lax.cond(chunk_ < LIN_CHUNKS, linear_flush, overflow_flush, 0)
        m_up8 = (m_ + 7) & ~7
        wtot_new = jnp.where(chunk_ < LIN_CHUNKS, wtot_ + m_up8, wtot_)
        return 0, chunk_ + 1, wtot_new

      def extract_vector(vec_i, b, carry, vbuf, from_tail):
        m, chunk, wtot = carry
        uvec = idx_v[pl.ds(vec_i * 16, 16)]
        pvec = bpos_v[pl.ds(vec_i * 16, 16)]
        gidx = vec_i * 16 + lanes
        if from_tail:
          match = (gidx < nmine) & (uvec >= ts)
        else:
          match = (gidx < nmine) & (lax.shift_right_logical(uvec, 7) == b)
        mi = match.astype(jnp.int32)
        pc = _lane0(plsc.all_reduce_population_count(match))

        @pl.when(pc > 0)
        def _do():
          slot_v = m + plsc.cumsum(mi) - mi
          lslot_v = (chunk & 1) * 128 + slot_v
          if from_tail:
            uloc_v = uvec - ts
          else:
            uloc_v = uvec & 127
          plsc.store_scatter(
              lpos_v,
              [jnp.full((LANES,), chunk & 1, jnp.int32), slot_v],
              pvec, mask=match)
          for kd in range(DIM):
            fk = (lanes + kd) & (DIM - 1)
            if from_tail:
              val = plsc.load_gather(vbuf, [uloc_v, fk], mask=match)
            else:
              val = plsc.load_gather(vbuf, [fk, uloc_v], mask=match)
            plsc.store_scatter(lrows, [lslot_v, fk], val, mask=match)

        return lax.cond(m + pc >= FLUSH_AT, flush, lambda c: c,
                        (m + pc, chunk, wtot))

      def scan_block(b, vbuf, carry):
        reg = jnp.minimum(
            lax.shift_right_logical(b - blk0, shift), NREG - 1)
        rs = jnp.sum(jnp.where(lanes == reg, starts0_v, 0))
        re = rs + jnp.sum(jnp.where(lanes == reg, cnts_v, 0))

        def vloop(v, c_):
          return extract_vector(v, b, c_, vbuf, from_tail=False)
        return lax.fori_loop(rs >> 4, (re + 15) >> 4, vloop, carry,
                             unroll=False)

      def start_copy(b, o):
        return pltpu.async_copy(
            tbl_hbm.at[:, pl.ds(b * 128, 128)], vbufs[o], sems[o])

      def wait_copy(o):
        pltpu.make_async_copy(tbl_hbm.at[:, pl.ds(0, 128)], vbufs[o],
                              sems[o]).wait()

      with jax.named_scope("ph_sweep_" + sfx):
        for o in range(NBUF - 1):
          @pl.when(blk0 + o < blk1)
          def _prime(o=o):
            start_copy(blk0 + o, o)

        def ring_body(q, carry):
          for o in range(NBUF):
            b = blk0 + q * NBUF + o

            def process(c_, b=b, o=o):
              wait_copy(o)

              @pl.when(b + NBUF - 1 < blk1)
              def _prefetch():
                start_copy(b + NBUF - 1, (o + NBUF - 1) % NBUF)

              return scan_block(b, vbufs[o], c_)

            carry = lax.cond(b < blk1, process, lambda c_: c_, carry)
          return carry

        carry = lax.fori_loop(0, (blk1 - blk0 + NBUF - 1) // NBUF,
                              ring_body, (0, 0, 0), unroll=False)

      with jax.named_scope("ph_tail_" + sfx):
        @pl.when(is_last)
        def _tail_copy():
          pltpu.sync_copy(tail_hbm,
                          vbufs[0].at[pl.ds(0, tailn), pl.ds(0, 128)])

        def tail_loop(v, c_):
          return extract_vector(v, 0, c_, vbufs[0], from_tail=True)
        carry = lax.cond(
            is_last,
            lambda c_: lax.fori_loop(0, nvec, tail_loop, c_, unroll=False),
            lambda c_: c_,
            carry)

        m_fin, chunk_fin, wtot_fin = carry

        @pl.when(m_fin > 0)
        def _final_flush():
          flush((m_fin, chunk_fin, wtot_fin))

        total_chunks = chunk_fin + jnp.where(m_fin > 0, 1, 0)
        lin_end = wtot_fin + jnp.where(
            (m_fin > 0) & (chunk_fin < LIN_CHUNKS), m_fin, 0)

        @pl.when(total_chunks >= 1)
        def _final_drain():
          drain_one()

        for t in range(128 // 16):
          lpos_v[0, pl.ds(t * 16, 16)] = safe_pos
        r8up = (lin_end + 7) & ~7
        pltpu.sync_copy(
            lpos_v.at[0],
            pmap_hbm.at[pl.ds(pl.multiple_of(myspan + r8up, 8), 128)])
        n_fill = lax.shift_right_logical(SPAN - r8up, 7)

        def fill_body(kf, _f):
          pltpu.sync_copy(
              lpos_v.at[0],
              pmap_hbm.at[pl.ds(
                  pl.multiple_of(myspan + SPAN - 128 * (kf + 1), 8), 128)])
          return _f
        lax.fori_loop(0, n_fill, fill_body, 0, unroll=False)

    run_table(U_SIZE, users_hbm, ut_hbm, tail_u_hbm, rows_u_hbm,
              pmap_u_hbm, "u")
    run_table(I_SIZE, items_hbm, it_hbm, tail_i_hbm, rows_i_hbm,
              pmap_i_hbm, "i")

  return k


def _make_dot():
  mesh = plsc.VectorSubcoreMesh(core_axis_name="c", subcore_axis_name="s")
  chunk = 128
  n_chunks = ROWS_PER_WORKER // chunk
  nmap_vecs = OVERFLOW_BASE // 16

  @functools.partial(
      pl.kernel,
      mesh=mesh,
      out_type=jax.ShapeDtypeStruct((BATCH,), jnp.float32),
      compiler_params=_COMPILER_PARAMS,
      scratch_types=[
          pltpu.VMEM((OVERFLOW_BASE,), jnp.int32),
          pltpu.VMEM((ROWS_PER_WORKER,), jnp.int32),
          pltpu.VMEM((ROWS_PER_WORKER,), jnp.int32),
          pltpu.VMEM((chunk, 2 * DIM), jnp.float32),
          pltpu.VMEM((chunk, 2 * DIM), jnp.float32),
          pltpu.VMEM((ROWS_PER_WORKER,), jnp.float32),
          pltpu.SemaphoreType.DMA,
      ],
  )
  def k(rows_u_hbm, pmap_u_hbm, rows_i_hbm, pmap_i_hbm, out_hbm,
        pbuf, locu_v, loci_v, ubuf, ibuf, out_v, sem):
    wid = lax.axis_index("s") * NUM_CORES + lax.axis_index("c")
    base = wid * ROWS_PER_WORKER
    lanes = lax.iota(jnp.int32, LANES)

    def invert(pmap_hbm, loc_v):
      pltpu.sync_copy(pmap_hbm.at[pl.ds(0, OVERFLOW_BASE)], pbuf)
      for t in range(ROWS_PER_WORKER // 16):
        loc_v[pl.ds(t * 16, 16)] = (
            OVERFLOW_BASE + base + t * 16 + lanes)

      def scan_body(v, _):
        posv = pbuf[pl.ds(v * 16, 16)]
        mask = (posv >= base) & (posv < base + ROWS_PER_WORKER)
        plsc.store_scatter(loc_v, [posv - base], v * 16 + lanes, mask=mask)
        return _
      lax.fori_loop(0, nmap_vecs, scan_body, 0, unroll=False)

    with jax.named_scope("ph_invert_u"):
      invert(pmap_u_hbm, locu_v)
    with jax.named_scope("ph_invert_i"):
      invert(pmap_i_hbm, loci_v)

    with jax.named_scope("ph_dot"):
      def chunk_body(c, _):
        cu = pltpu.async_copy(
            rows_u_hbm.at[locu_v.at[pl.ds(c * chunk, chunk)]], ubuf, sem)
        ci = pltpu.async_copy(
            rows_i_hbm.at[loci_v.at[pl.ds(c * chunk, chunk)]], ibuf, sem)
        cu.wait()
        ci.wait()

        def group_body(g, _g):
          j_vec = g * 16 + lanes
          acc = jnp.zeros((16,), jnp.float32)
          for d in range(DIM):
            col = (lanes + d) & (DIM - 1)
            ug = plsc.load_gather(ubuf, [j_vec, col])
            ig = plsc.load_gather(ibuf, [j_vec, col])
            acc = acc + ug * ig
          out_v[pl.ds(c * chunk + g * 16, 16)] = acc
          return _g
        lax.fori_loop(0, chunk // 16, group_body, 0, unroll=False)
        return _

      lax.fori_loop(0, n_chunks, chunk_body, 0, unroll=False)
      pltpu.sync_copy(out_v, out_hbm.at[pl.ds(base, ROWS_PER_WORKER)])

  return k


_extract = _make_extract()
_dot = _make_dot()

_U_TS = (U_SIZE // 128) * 128
_I_TS = (I_SIZE // 128) * 128


@jax.jit
def kernel(users, items, user_emb, item_emb):
  tail_u = jnp.pad(user_emb[_U_TS:], ((0, 0), (0, DIM)))
  tail_i = jnp.pad(item_emb[_I_TS:], ((0, 0), (0, DIM)))
  rows_u, pmap_u, rows_i, pmap_i = _extract(
      users, items, user_emb.T, item_emb.T, tail_u, tail_i)
  return _dot(rows_u, pmap_u, rows_i, pmap_i)


# speedup vs baseline: 2.1155x; 1.0268x over previous
import functools

import jax
import jax.numpy as jnp
from jax import lax
from jax.experimental import pallas as pl
from jax.experimental.pallas import tpu as pltpu
from jax.experimental.pallas import tpu_sc as plsc

U_SIZE = 1000000
I_SIZE = 100000
DIM = 64
BATCH = 16384

NUM_CORES = 2
NUM_SUBCORES = 16
NUM_WORKERS = NUM_CORES * NUM_SUBCORES
ROWS_PER_WORKER = BATCH // NUM_WORKERS
SPAN = 1024
LIN_CHUNKS = 7
OVERFLOW_BASE = NUM_WORKERS * SPAN
STAGE_ROWS = OVERFLOW_BASE + BATCH + 128
POSMAP_LEN = OVERFLOW_BASE + 128
CAP = BATCH
NIDX_VECS = BATCH // 16
LANES = 16
NREG = 16
FLUSH_AT = 113
NBUF = 4

_COMPILER_PARAMS = pltpu.CompilerParams(
    needs_layout_passes=False, use_tc_tiling_on_sc=True,
    disable_bounds_checks=True, disable_semaphore_checks=True)


def _lane0(v):
  return lax.squeeze(lax.slice(v, (0,), (1,)), dimensions=(0,))


def _lane(v, i):
  return lax.squeeze(lax.slice(v, (i,), (i + 1,)), dimensions=(0,))


def _make_extract():
  mesh = plsc.VectorSubcoreMesh(core_axis_name="c", subcore_axis_name="s")

  @functools.partial(
      pl.kernel,
      mesh=mesh,
      out_type=(jax.ShapeDtypeStruct((STAGE_ROWS, 2 * DIM), jnp.float32),
                jax.ShapeDtypeStruct((POSMAP_LEN,), jnp.int32),
                jax.ShapeDtypeStruct((STAGE_ROWS, 2 * DIM), jnp.float32),
                jax.ShapeDtypeStruct((POSMAP_LEN,), jnp.int32)),
      compiler_params=_COMPILER_PARAMS,
      scratch_types=[
          pltpu.VMEM((BATCH,), jnp.int32),
          pltpu.VMEM((CAP + 16,), jnp.int32),
          pltpu.VMEM((CAP + 16,), jnp.int32),
          pltpu.VMEM((CAP,), jnp.int32),
          [pltpu.VMEM((64, 128), jnp.float32) for _ in range(NBUF)],
          pltpu.VMEM((144, 2 * DIM), jnp.float32),
          pltpu.VMEM((2, 128), jnp.int32),
          [pltpu.SemaphoreType.DMA for _ in range(NBUF)],
          pltpu.SemaphoreType.DMA,
      ],
  )
  def k(users_hbm, items_hbm, ut_hbm, it_hbm, tail_u_hbm, tail_i_hbm,
        rows_u_hbm, pmap_u_hbm, rows_i_hbm, pmap_i_hbm,
        idx_v, myu_v, mypos_v, bpos_v, vbufs, lrows, lpos_v, sems, semw):
    wid = lax.axis_index("s") * NUM_CORES + lax.axis_index("c")
    is_last = wid == NUM_WORKERS - 1
    lanes = lax.iota(jnp.int32, LANES)
    safe_pos = jnp.full((LANES,), BATCH, jnp.int32)

    for j in range(2):
      for t in range(128 // 16):
        lpos_v[j, pl.ds(t * 16, 16)] = safe_pos

    def run_table(n_rows, idx_hbm, tbl_hbm, tail_hbm, rows_hbm, pmap_hbm,
                  sfx):
      nb = n_rows // 128
      ts = nb * 128
      tailn = n_rows - ts
      max_wblocks = -(-nb // NUM_WORKERS) + 1
      shift = max(0, (-(-max_wblocks // NREG) - 1).bit_length())
      blk0 = (wid * nb) >> 5
      blk1 = ((wid + 1) * nb) >> 5

      with jax.named_scope("ph_init_" + sfx):
        pltpu.sync_copy(idx_hbm, idx_v)

      with jax.named_scope("ph_filter_" + sfx):
        def fbody(i, ptr_v):
          ptr = _lane0(ptr_v)
          uvec = idx_v[pl.ds(i * 16, 16)]
          q = lax.shift_right_logical(uvec, 7)
          m = (q >= blk0) & (q < blk1)
          m = m | (is_last & (uvec >= ts))
          plsc.store_compressed(myu_v.at[pl.ds(ptr, 16)], uvec, mask=m)
          plsc.store_compressed(mypos_v.at[pl.ds(ptr, 16)],
                                i * 16 + lanes, mask=m)
          return ptr_v + plsc.all_reduce_population_count(m)
        nmine_v = lax.fori_loop(0, NIDX_VECS, fbody,
                                jnp.zeros((LANES,), jnp.int32),
                                unroll=False)
        nmine = _lane0(nmine_v)
        nvec = (nmine + 15) >> 4

      def region_of(uvec):
        r = lax.shift_right_logical(
            lax.shift_right_logical(uvec, 7) - blk0, shift)
        return jnp.minimum(r, NREG - 1)

      def cbody(v, cnts):
        uvec = myu_v[pl.ds(v * 16, 16)]
        valid = (v * 16 + lanes) < nmine
        r = region_of(uvec)
        for reg in range(NREG):
          pc = plsc.all_reduce_population_count((r == reg) & valid)
          cnts = cnts + jnp.where(lanes == reg, pc, 0)
        return cnts
      with jax.named_scope("ph_bucketA_" + sfx):
        cnts_v = lax.fori_loop(0, nvec, cbody,
                               jnp.zeros((LANES,), jnp.int32),
                               unroll=False)
        starts0_v = plsc.cumsum(cnts_v) - cnts_v

      def bbody(v, starts):
        uvec = myu_v[pl.ds(v * 16, 16)]
        pvec = mypos_v[pl.ds(v * 16, 16)]
        valid = (v * 16 + lanes) < nmine
        r = region_of(uvec)
        for reg in range(NREG):
          m = (r == reg) & valid
          ptr = _lane(starts, reg)
          plsc.store_compressed(idx_v.at[pl.ds(ptr, 16)], uvec, mask=m)
          plsc.store_compressed(bpos_v.at[pl.ds(ptr, 16)], pvec, mask=m)
          pc = plsc.all_reduce_population_count(m)
          starts = starts + jnp.where(lanes == reg, pc, 0)
        return starts
      with jax.named_scope("ph_bucketB_" + sfx):
        lax.fori_loop(0, nvec, bbody, starts0_v, unroll=False)

      myspan = wid * SPAN

      def flush(c):
        m_, chunk_, wtot_ = c

        for t in range(128 // 16):
          plsc.store_scatter(lpos_v,
                             [jnp.full((LANES,), 0, jnp.int32),
                              t * 16 + lanes],
                             safe_pos, mask=(t * 16 + lanes) >= m_)

        def linear_flush(_):
          off = pl.multiple_of(myspan + wtot_, 8)
          pltpu.async_copy(lrows.at[pl.ds(0, 128)],
                           rows_hbm.at[pl.ds(off, 128)], semw).wait()
          pltpu.sync_copy(lpos_v.at[0], pmap_hbm.at[pl.ds(off, 128)])
          return 0

        def overflow_flush(_):
          for t in range(128 // 16):
            lpos_v[0, pl.ds(t * 16, 16)] = (
                lpos_v[0, pl.ds(t * 16, 16)] + OVERFLOW_BASE)
          pltpu.async_copy(lrows.at[pl.ds(0, 128)],
                           rows_hbm.at[lpos_v.at[0]], semw).wait()
          return 0

        lax.cond(chunk_ < LIN_CHUNKS, linear_flush, overflow_flush, 0)
        m_up8 = (m_ + 7) & ~7
        wtot_new = jnp.where(chunk_ < LIN_CHUNKS, wtot_ + m_up8, wtot_)
        return 0, chunk_ + 1, wtot_new

      def extract_vector(vec_i, b, carry, vbuf, from_tail):
        m, chunk, wtot = carry
        uvec = idx_v[pl.ds(vec_i * 16, 16)]
        pvec = bpos_v[pl.ds(vec_i * 16, 16)]
        gidx = vec_i * 16 + lanes
        if from_tail:
          match = (gidx < nmine) & (uvec >= ts)
        else:
          match = (gidx < nmine) & (lax.shift_right_logical(uvec, 7) == b)
        mi = match.astype(jnp.int32)
        pc = _lane0(plsc.all_reduce_population_count(match))

        @pl.when(pc > 0)
        def _do():
          slot_v = m + plsc.cumsum(mi) - mi
          lslot_v = slot_v
          if from_tail:
            uloc_v = uvec - ts
          else:
            uloc_v = uvec & 127
          plsc.store_scatter(
              lpos_v,
              [jnp.full((LANES,), 0, jnp.int32), slot_v],
              pvec, mask=match)
          for kd in range(DIM):
            fk = (lanes + kd) & (DIM - 1)
            if from_tail:
              val = plsc.load_gather(vbuf, [uloc_v, fk], mask=match)
            else:
              val = plsc.load_gather(vbuf, [fk, uloc_v], mask=match)
            plsc.store_scatter(lrows, [lslot_v, fk], val, mask=match)

        return lax.cond(m + pc >= FLUSH_AT, flush, lambda c: c,
                        (m + pc, chunk, wtot))

      def scan_block(b, vbuf, carry):
        reg = jnp.minimum(
            lax.shift_right_logical(b - blk0, shift), NREG - 1)
        rs = jnp.sum(jnp.where(lanes == reg, starts0_v, 0))
        re = rs + jnp.sum(jnp.where(lanes == reg, cnts_v, 0))

        def vloop(v, c_):
          return extract_vector(v, b, c_, vbuf, from_tail=False)
        return lax.fori_loop(rs >> 4, (re + 15) >> 4, vloop, carry,
                             unroll=False)

      def start_copy(b, o):
        return pltpu.async_copy(
            tbl_hbm.at[:, pl.ds(b * 128, 128)], vbufs[o], sems[o])

      def wait_copy(o):
        pltpu.make_async_copy(tbl_hbm.at[:, pl.ds(0, 128)], vbufs[o],
                              sems[o]).wait()

      with jax.named_scope("ph_sweep_" + sfx):
        for o in range(NBUF - 1):
          @pl.when(blk0 + o < blk1)
          def _prime(o=o):
            start_copy(blk0 + o, o)

        def ring_body(q, carry):
          for o in range(NBUF):
            b = blk0 + q * NBUF + o

            def process(c_, b=b, o=o):
              wait_copy(o)

              @pl.when(b + NBUF - 1 < blk1)
              def _prefetch():
                start_copy(b + NBUF - 1, (o + NBUF - 1) % NBUF)

              return scan_block(b, vbufs[o], c_)

            carry = lax.cond(b < blk1, process, lambda c_: c_, carry)
          return carry

        carry = lax.fori_loop(0, (blk1 - blk0 + NBUF - 1) // NBUF,
                              ring_body, (0, 0, 0), unroll=False)

      with jax.named_scope("ph_tail_" + sfx):
        @pl.when(is_last)
        def _tail_copy():
          pltpu.sync_copy(tail_hbm,
                          vbufs[0].at[pl.ds(0, tailn), pl.ds(0, 128)])

        def tail_loop(v, c_):
          return extract_vector(v, 0, c_, vbufs[0], from_tail=True)
        carry = lax.cond(
            is_last,
            lambda c_: lax.fori_loop(0, nvec, tail_loop, c_, unroll=False),
            lambda c_: c_,
            carry)

        m_fin, chunk_fin, wtot_fin = carry

        @pl.when(m_fin > 0)
        def _final_flush():
          flush((m_fin, chunk_fin, wtot_fin))

        lin_end = wtot_fin + jnp.where(
            (m_fin > 0) & (chunk_fin < LIN_CHUNKS), m_fin, 0)

        for t in range(128 // 16):
          lpos_v[0, pl.ds(t * 16, 16)] = safe_pos
        r8up = (lin_end + 7) & ~7
        pltpu.sync_copy(
            lpos_v.at[0],
            pmap_hbm.at[pl.ds(pl.multiple_of(myspan + r8up, 8), 128)])
        n_fill = lax.shift_right_logical(SPAN - r8up, 7)

        def fill_body(kf, _f):
          pltpu.sync_copy(
              lpos_v.at[0],
              pmap_hbm.at[pl.ds(
                  pl.multiple_of(myspan + SPAN - 128 * (kf + 1), 8), 128)])
          return _f
        lax.fori_loop(0, n_fill, fill_body, 0, unroll=False)

    run_table(U_SIZE, users_hbm, ut_hbm, tail_u_hbm, rows_u_hbm,
              pmap_u_hbm, "u")
    run_table(I_SIZE, items_hbm, it_hbm, tail_i_hbm, rows_i_hbm,
              pmap_i_hbm, "i")

  return k


def _make_dot():
  mesh = plsc.VectorSubcoreMesh(core_axis_name="c", subcore_axis_name="s")
  chunk = 128
  n_chunks = ROWS_PER_WORKER // chunk
  nmap_vecs = OVERFLOW_BASE // 16

  @functools.partial(
      pl.kernel,
      mesh=mesh,
      out_type=jax.ShapeDtypeStruct((BATCH,), jnp.float32),
      compiler_params=_COMPILER_PARAMS,
      scratch_types=[
          pltpu.VMEM((OVERFLOW_BASE,), jnp.int32),
          pltpu.VMEM((ROWS_PER_WORKER,), jnp.int32),
          pltpu.VMEM((ROWS_PER_WORKER,), jnp.int32),
          pltpu.VMEM((chunk, 2 * DIM), jnp.float32),
          pltpu.VMEM((chunk, 2 * DIM), jnp.float32),
          pltpu.VMEM((ROWS_PER_WORKER,), jnp.float32),
          pltpu.SemaphoreType.DMA,
      ],
  )
  def k(rows_u_hbm, pmap_u_hbm, rows_i_hbm, pmap_i_hbm, out_hbm,
        pbuf, locu_v, loci_v, ubuf, ibuf, out_v, sem):
    wid = lax.axis_index("s") * NUM_CORES + lax.axis_index("c")
    base = wid * ROWS_PER_WORKER
    lanes = lax.iota(jnp.int32, LANES)

    def invert(pmap_hbm, loc_v):
      pltpu.sync_copy(pmap_hbm.at[pl.ds(0, OVERFLOW_BASE)], pbuf)
      for t in range(ROWS_PER_WORKER // 16):
        loc_v[pl.ds(t * 16, 16)] = (
            OVERFLOW_BASE + base + t * 16 + lanes)

      def scan_body(v, _):
        posv = pbuf[pl.ds(v * 16, 16)]
        mask = (posv >= base) & (posv < base + ROWS_PER_WORKER)
        plsc.store_scatter(loc_v, [posv - base], v * 16 + lanes, mask=mask)
        return _
      lax.fori_loop(0, nmap_vecs, scan_body, 0, unroll=4)

    with jax.named_scope("ph_invert_u"):
      invert(pmap_u_hbm, locu_v)
    with jax.named_scope("ph_invert_i"):
      invert(pmap_i_hbm, loci_v)

    with jax.named_scope("ph_dot"):
      def chunk_body(c, _):
        cu = pltpu.async_copy(
            rows_u_hbm.at[locu_v.at[pl.ds(c * chunk, chunk)]], ubuf, sem)
        ci = pltpu.async_copy(
            rows_i_hbm.at[loci_v.at[pl.ds(c * chunk, chunk)]], ibuf, sem)
        cu.wait()
        ci.wait()

        def group_body(g, _g):
          j_vec = g * 16 + lanes
          acc = jnp.zeros((16,), jnp.float32)
          for d in range(DIM):
            col = (lanes + d) & (DIM - 1)
            ug = plsc.load_gather(ubuf, [j_vec, col])
            ig = plsc.load_gather(ibuf, [j_vec, col])
            acc = acc + ug * ig
          out_v[pl.ds(c * chunk + g * 16, 16)] = acc
          return _g
        lax.fori_loop(0, chunk // 16, group_body, 0, unroll=False)
        return _

      lax.fori_loop(0, n_chunks, chunk_body, 0, unroll=False)
      pltpu.sync_copy(out_v, out_hbm.at[pl.ds(base, ROWS_PER_WORKER)])

  return k


_extract = _make_extract()
_dot = _make_dot()

_U_TS = (U_SIZE // 128) * 128
_I_TS = (I_SIZE // 128) * 128


@jax.jit
def kernel(users, items, user_emb, item_emb):
  tail_u = jnp.pad(user_emb[_U_TS:], ((0, 0), (0, DIM)))
  tail_i = jnp.pad(item_emb[_I_TS:], ((0, 0), (0, DIM)))
  rows_u, pmap_u, rows_i, pmap_i = _extract(
      users, items, user_emb.T, item_emb.T, tail_u, tail_i)
  return _dot(rows_u, pmap_u, rows_i, pmap_i)


# speedup vs baseline: 2.1232x; 1.0037x over previous
import functools

import jax
import jax.numpy as jnp
from jax import lax
from jax.experimental import pallas as pl
from jax.experimental.pallas import tpu as pltpu
from jax.experimental.pallas import tpu_sc as plsc

U_SIZE = 1000000
I_SIZE = 100000
DIM = 64
BATCH = 16384

NUM_CORES = 2
NUM_SUBCORES = 16
NUM_WORKERS = NUM_CORES * NUM_SUBCORES
ROWS_PER_WORKER = BATCH // NUM_WORKERS
SPAN = 1024
LIN_CHUNKS = 7
OVERFLOW_BASE = NUM_WORKERS * SPAN
STAGE_ROWS = OVERFLOW_BASE + BATCH + 128
POSMAP_LEN = OVERFLOW_BASE + 128
CAP = BATCH
NIDX_VECS = BATCH // 16
LANES = 16
NREG = 16
FLUSH_AT = 113
NBUF = 4

_COMPILER_PARAMS = pltpu.CompilerParams(
    needs_layout_passes=False, use_tc_tiling_on_sc=True,
    disable_bounds_checks=True, disable_semaphore_checks=True)


def _lane0(v):
  return lax.squeeze(lax.slice(v, (0,), (1,)), dimensions=(0,))


def _lane(v, i):
  return lax.squeeze(lax.slice(v, (i,), (i + 1,)), dimensions=(0,))


def _make_extract():
  mesh = plsc.VectorSubcoreMesh(core_axis_name="c", subcore_axis_name="s")

  @functools.partial(
      pl.kernel,
      mesh=mesh,
      out_type=(jax.ShapeDtypeStruct((STAGE_ROWS, 2 * DIM), jnp.float32),
                jax.ShapeDtypeStruct((POSMAP_LEN,), jnp.int32),
                jax.ShapeDtypeStruct((STAGE_ROWS, 2 * DIM), jnp.float32),
                jax.ShapeDtypeStruct((POSMAP_LEN,), jnp.int32)),
      compiler_params=_COMPILER_PARAMS,
      scratch_types=[
          pltpu.VMEM((BATCH,), jnp.int32),
          pltpu.VMEM((CAP + 16,), jnp.int32),
          pltpu.VMEM((CAP + 16,), jnp.int32),
          pltpu.VMEM((CAP,), jnp.int32),
          [pltpu.VMEM((64, 128), jnp.float32) for _ in range(NBUF)],
          pltpu.VMEM((144, 2 * DIM), jnp.float32),
          pltpu.VMEM((2, 128), jnp.int32),
          [pltpu.SemaphoreType.DMA for _ in range(NBUF)],
          pltpu.SemaphoreType.DMA,
      ],
  )
  def k(users_hbm, items_hbm, ut_hbm, it_hbm, tail_u_hbm, tail_i_hbm,
        rows_u_hbm, pmap_u_hbm, rows_i_hbm, pmap_i_hbm,
        idx_v, myu_v, mypos_v, bpos_v, vbufs, lrows, lpos_v, sems, semw):
    wid = lax.axis_index("s") * NUM_CORES + lax.axis_index("c")
    is_last = wid == NUM_WORKERS - 1
    lanes = lax.iota(jnp.int32, LANES)
    safe_pos = jnp.full((LANES,), BATCH, jnp.int32)

    for j in range(2):
      for t in range(128 // 16):
        lpos_v[j, pl.ds(t * 16, 16)] = safe_pos

    def run_table(n_rows, idx_hbm, tbl_hbm, tail_hbm, rows_hbm, pmap_hbm,
                  sfx):
      nb = n_rows // 128
      ts = nb * 128
      tailn = n_rows - ts
      max_wblocks = -(-nb // NUM_WORKERS) + 1
      shift = max(0, (-(-max_wblocks // NREG) - 1).bit_length())
      blk0 = (wid * nb) >> 5
      blk1 = ((wid + 1) * nb) >> 5

      with jax.named_scope("ph_init_" + sfx):
        pltpu.sync_copy(idx_hbm, idx_v)

      with jax.named_scope("ph_filter_" + sfx):
        def fbody(i, ptr_v):
          ptr = _lane0(ptr_v)
          uvec = idx_v[pl.ds(i * 16, 16)]
          q = lax.shift_right_logical(uvec, 7)
          m = (q >= blk0) & (q < blk1)
          m = m | (is_last & (uvec >= ts))
          plsc.store_compressed(myu_v.at[pl.ds(ptr, 16)], uvec, mask=m)
          plsc.store_compressed(mypos_v.at[pl.ds(ptr, 16)],
                                i * 16 + lanes, mask=m)
          return ptr_v + plsc.all_reduce_population_count(m)
        nmine_v = lax.fori_loop(0, NIDX_VECS, fbody,
                                jnp.zeros((LANES,), jnp.int32),
                                unroll=4)
        nmine = _lane0(nmine_v)
        nvec = (nmine + 15) >> 4

      def region_of(uvec):
        r = lax.shift_right_logical(
            lax.shift_right_logical(uvec, 7) - blk0, shift)
        return jnp.minimum(r, NREG - 1)

      def cbody(v, cnts):
        uvec = myu_v[pl.ds(v * 16, 16)]
        valid = (v * 16 + lanes) < nmine
        r = region_of(uvec)
        for reg in range(NREG):
          pc = plsc.all_reduce_population_count((r == reg) & valid)
          cnts = cnts + jnp.where(lanes == reg, pc, 0)
        return cnts
      with jax.named_scope("ph_bucketA_" + sfx):
        cnts_v = lax.fori_loop(0, nvec, cbody,
                               jnp.zeros((LANES,), jnp.int32),
                               unroll=False)
        starts0_v = plsc.cumsum(cnts_v) - cnts_v

      def bbody(v, starts):
        uvec = myu_v[pl.ds(v * 16, 16)]
        pvec = mypos_v[pl.ds(v * 16, 16)]
        valid = (v * 16 + lanes) < nmine
        r = region_of(uvec)
        for reg in range(NREG):
          m = (r == reg) & valid
          ptr = _lane(starts, reg)
          plsc.store_compressed(idx_v.at[pl.ds(ptr, 16)], uvec, mask=m)
          plsc.store_compressed(bpos_v.at[pl.ds(ptr, 16)], pvec, mask=m)
          pc = plsc.all_reduce_population_count(m)
          starts = starts + jnp.where(lanes == reg, pc, 0)
        return starts
      with jax.named_scope("ph_bucketB_" + sfx):
        lax.fori_loop(0, nvec, bbody, starts0_v, unroll=False)

      myspan = wid * SPAN

      def flush(c):
        m_, chunk_, wtot_ = c

        for t in range(128 // 16):
          plsc.store_scatter(lpos_v,
                             [jnp.full((LANES,), 0, jnp.int32),
                              t * 16 + lanes],
                             safe_pos, mask=(t * 16 + lanes) >= m_)

        def linear_flush(_):
          off = pl.multiple_of(myspan + wtot_, 8)
          pltpu.async_copy(lrows.at[pl.ds(0, 128)],
                           rows_hbm.at[pl.ds(off, 128)], semw).wait()
          pltpu.sync_copy(lpos_v.at[0], pmap_hbm.at[pl.ds(off, 128)])
          return 0

        def overflow_flush(_):
          for t in range(128 // 16):
            lpos_v[0, pl.ds(t * 16, 16)] = (
                lpos_v[0, pl.ds(t * 16, 16)] + OVERFLOW_BASE)
          pltpu.async_copy(lrows.at[pl.ds(0, 128)],
                           rows_hbm.at[lpos_v.at[0]], semw).wait()
          return 0

        lax.cond(chunk_ < LIN_CHUNKS, linear_flush, overflow_flush, 0)
        m_up8 = (m_ + 7) & ~7
        wtot_new = jnp.where(chunk_ < LIN_CHUNKS, wtot_ + m_up8, wtot_)
        return 0, chunk_ + 1, wtot_new

      def extract_vector(vec_i, b, carry, vbuf, from_tail):
        m, chunk, wtot = carry
        uvec = idx_v[pl.ds(vec_i * 16, 16)]
        pvec = bpos_v[pl.ds(vec_i * 16, 16)]
        gidx = vec_i * 16 + lanes
        if from_tail:
          match = (gidx < nmine) & (uvec >= ts)
        else:
          match = (gidx < nmine) & (lax.shift_right_logical(uvec, 7) == b)
        mi = match.astype(jnp.int32)
        pc = _lane0(plsc.all_reduce_population_count(match))

        @pl.when(pc > 0)
        def _do():
          slot_v = m + plsc.cumsum(mi) - mi
          lslot_v = slot_v
          if from_tail:
            uloc_v = uvec - ts
          else:
            uloc_v = uvec & 127
          plsc.store_scatter(
              lpos_v,
              [jnp.full((LANES,), 0, jnp.int32), slot_v],
              pvec, mask=match)
          for kd in range(DIM):
            fk = (lanes + kd) & (DIM - 1)
            if from_tail:
              val = plsc.load_gather(vbuf, [uloc_v, fk], mask=match)
            else:
              val = plsc.load_gather(vbuf, [fk, uloc_v], mask=match)
            plsc.store_scatter(lrows, [lslot_v, fk], val, mask=match)

        return lax.cond(m + pc >= FLUSH_AT, flush, lambda c: c,
                        (m + pc, chunk, wtot))

      def scan_block(b, vbuf, carry):
        reg = jnp.minimum(
            lax.shift_right_logical(b - blk0, shift), NREG - 1)
        rs = jnp.sum(jnp.where(lanes == reg, starts0_v, 0))
        re = rs + jnp.sum(jnp.where(lanes == reg, cnts_v, 0))

        def vloop(v, c_):
          return extract_vector(v, b, c_, vbuf, from_tail=False)
        return lax.fori_loop(rs >> 4, (re + 15) >> 4, vloop, carry,
                             unroll=False)

      def start_copy(b, o):
        return pltpu.async_copy(
            tbl_hbm.at[:, pl.ds(b * 128, 128)], vbufs[o], sems[o])

      def wait_copy(o):
        pltpu.make_async_copy(tbl_hbm.at[:, pl.ds(0, 128)], vbufs[o],
                              sems[o]).wait()

      with jax.named_scope("ph_sweep_" + sfx):
        for o in range(NBUF - 1):
          @pl.when(blk0 + o < blk1)
          def _prime(o=o):
            start_copy(blk0 + o, o)

        def ring_body(q, carry):
          for o in range(NBUF):
            b = blk0 + q * NBUF + o

            def process(c_, b=b, o=o):
              wait_copy(o)

              @pl.when(b + NBUF - 1 < blk1)
              def _prefetch():
                start_copy(b + NBUF - 1, (o + NBUF - 1) % NBUF)

              return scan_block(b, vbufs[o], c_)

            carry = lax.cond(b < blk1, process, lambda c_: c_, carry)
          return carry

        carry = lax.fori_loop(0, (blk1 - blk0 + NBUF - 1) // NBUF,
                              ring_body, (0, 0, 0), unroll=False)

      with jax.named_scope("ph_tail_" + sfx):
        @pl.when(is_last)
        def _tail_copy():
          pltpu.sync_copy(tail_hbm,
                          vbufs[0].at[pl.ds(0, tailn), pl.ds(0, 128)])

        def tail_loop(v, c_):
          return extract_vector(v, 0, c_, vbufs[0], from_tail=True)
        carry = lax.cond(
            is_last,
            lambda c_: lax.fori_loop(0, nvec, tail_loop, c_, unroll=False),
            lambda c_: c_,
            carry)

        m_fin, chunk_fin, wtot_fin = carry

        @pl.when(m_fin > 0)
        def _final_flush():
          flush((m_fin, chunk_fin, wtot_fin))

        lin_end = wtot_fin + jnp.where(
            (m_fin > 0) & (chunk_fin < LIN_CHUNKS), m_fin, 0)

        for t in range(128 // 16):
          lpos_v[0, pl.ds(t * 16, 16)] = safe_pos
        r8up = (lin_end + 7) & ~7
        pltpu.sync_copy(
            lpos_v.at[0],
            pmap_hbm.at[pl.ds(pl.multiple_of(myspan + r8up, 8), 128)])
        n_fill = lax.shift_right_logical(SPAN - r8up, 7)

        def fill_body(kf, _f):
          pltpu.sync_copy(
              lpos_v.at[0],
              pmap_hbm.at[pl.ds(
                  pl.multiple_of(myspan + SPAN - 128 * (kf + 1), 8), 128)])
          return _f
        lax.fori_loop(0, n_fill, fill_body, 0, unroll=False)

    run_table(U_SIZE, users_hbm, ut_hbm, tail_u_hbm, rows_u_hbm,
              pmap_u_hbm, "u")
    run_table(I_SIZE, items_hbm, it_hbm, tail_i_hbm, rows_i_hbm,
              pmap_i_hbm, "i")

  return k


def _make_dot():
  mesh = plsc.VectorSubcoreMesh(core_axis_name="c", subcore_axis_name="s")
  chunk = 128
  n_chunks = ROWS_PER_WORKER // chunk
  nmap_vecs = OVERFLOW_BASE // 16

  @functools.partial(
      pl.kernel,
      mesh=mesh,
      out_type=jax.ShapeDtypeStruct((BATCH,), jnp.float32),
      compiler_params=_COMPILER_PARAMS,
      scratch_types=[
          pltpu.VMEM((OVERFLOW_BASE,), jnp.int32),
          pltpu.VMEM((ROWS_PER_WORKER,), jnp.int32),
          pltpu.VMEM((ROWS_PER_WORKER,), jnp.int32),
          pltpu.VMEM((chunk, 2 * DIM), jnp.float32),
          pltpu.VMEM((chunk, 2 * DIM), jnp.float32),
          pltpu.VMEM((ROWS_PER_WORKER,), jnp.float32),
          pltpu.SemaphoreType.DMA,
      ],
  )
  def k(rows_u_hbm, pmap_u_hbm, rows_i_hbm, pmap_i_hbm, out_hbm,
        pbuf, locu_v, loci_v, ubuf, ibuf, out_v, sem):
    wid = lax.axis_index("s") * NUM_CORES + lax.axis_index("c")
    base = wid * ROWS_PER_WORKER
    lanes = lax.iota(jnp.int32, LANES)

    def invert(pmap_hbm, loc_v):
      pltpu.sync_copy(pmap_hbm.at[pl.ds(0, OVERFLOW_BASE)], pbuf)
      for t in range(ROWS_PER_WORKER // 16):
        loc_v[pl.ds(t * 16, 16)] = (
            OVERFLOW_BASE + base + t * 16 + lanes)

      def scan_body(v, _):
        posv = pbuf[pl.ds(v * 16, 16)]
        mask = (posv >= base) & (posv < base + ROWS_PER_WORKER)
        plsc.store_scatter(loc_v, [posv - base], v * 16 + lanes, mask=mask)
        return _
      lax.fori_loop(0, nmap_vecs, scan_body, 0, unroll=4)

    with jax.named_scope("ph_invert_u"):
      invert(pmap_u_hbm, locu_v)
    with jax.named_scope("ph_invert_i"):
      invert(pmap_i_hbm, loci_v)

    with jax.named_scope("ph_dot"):
      def chunk_body(c, _):
        cu = pltpu.async_copy(
            rows_u_hbm.at[locu_v.at[pl.ds(c * chunk, chunk)]], ubuf, sem)
        ci = pltpu.async_copy(
            rows_i_hbm.at[loci_v.at[pl.ds(c * chunk, chunk)]], ibuf, sem)
        cu.wait()
        ci.wait()

        def group_body(g, _g):
          j_vec = g * 16 + lanes
          acc = jnp.zeros((16,), jnp.float32)
          for d in range(DIM):
            col = (lanes + d) & (DIM - 1)
            ug = plsc.load_gather(ubuf, [j_vec, col])
            ig = plsc.load_gather(ibuf, [j_vec, col])
            acc = acc + ug * ig
          out_v[pl.ds(c * chunk + g * 16, 16)] = acc
          return _g
        lax.fori_loop(0, chunk // 16, group_body, 0, unroll=False)
        return _

      lax.fori_loop(0, n_chunks, chunk_body, 0, unroll=False)
      pltpu.sync_copy(out_v, out_hbm.at[pl.ds(base, ROWS_PER_WORKER)])

  return k


_extract = _make_extract()
_dot = _make_dot()

_U_TS = (U_SIZE // 128) * 128
_I_TS = (I_SIZE // 128) * 128


@jax.jit
def kernel(users, items, user_emb, item_emb):
  tail_u = jnp.pad(user_emb[_U_TS:], ((0, 0), (0, DIM)))
  tail_i = jnp.pad(item_emb[_I_TS:], ((0, 0), (0, DIM)))
  rows_u, pmap_u, rows_i, pmap_i = _extract(
      users, items, user_emb.T, item_emb.T, tail_u, tail_i)
  return _dot(rows_u, pmap_u, rows_i, pmap_i)
